# speedup vs baseline: 5.1714x; 5.1714x over previous
import functools

import jax
import jax.numpy as jnp
from jax import lax
from jax.experimental import pallas as pl
from jax.experimental.pallas import tpu as pltpu
from jax.experimental.pallas import tpu_sc as plsc

_VOCAB = 100000
_D = 128
_B = 4096
_H = 200
_N = _B * _H

_NC = 2
_NS = 16
_NW = _NC * _NS
_PER_W = _N // _NW
_CH = 128
_NCH = _PER_W // _CH

_mesh = plsc.VectorSubcoreMesh(core_axis_name="c", subcore_axis_name="s")


@functools.partial(
    pl.kernel,
    mesh=_mesh,
    out_type=jax.ShapeDtypeStruct((_N, _D), jnp.float32),
    scratch_types=[
        pltpu.VMEM((_CH,), jnp.int32),
        pltpu.VMEM((_CH, _D), jnp.float32),
        pltpu.SemaphoreType.DMA,
    ],
)
def _gather(x_hbm, table_hbm, out_hbm, idx_v, rows_v, sem):
    wid = lax.axis_index("s") * _NC + lax.axis_index("c")
    base = wid * _PER_W

    def body(i, carry):
        off = base + i * _CH
        pltpu.sync_copy(x_hbm.at[pl.ds(off, _CH)], idx_v)
        pltpu.async_copy(table_hbm.at[idx_v], rows_v, sem).wait()
        pltpu.sync_copy(rows_v, out_hbm.at[pl.ds(off, _CH)])
        return carry

    lax.fori_loop(0, _NCH, body, 0)


def kernel(x, table):
    flat = x.reshape(_N).astype(jnp.int32)
    out = _gather(flat, table)
    return out.reshape(_B, _H, _D)


# speedup vs baseline: 9.2138x; 1.7817x over previous
import functools

import jax
import jax.numpy as jnp
from jax import lax
from jax.experimental import pallas as pl
from jax.experimental.pallas import tpu as pltpu
from jax.experimental.pallas import tpu_sc as plsc

_VOCAB = 100000
_D = 128
_B = 4096
_H = 200
_N = _B * _H

_NC = 2
_NS = 16
_NW = _NC * _NS
_PER_W = _N // _NW
_CH = 128
_NCH = _PER_W // _CH
_G = 2
_NB = 2
_ROWS = _G * _CH
_NG = _NCH // _G

_mesh = plsc.VectorSubcoreMesh(core_axis_name="c", subcore_axis_name="s")


@functools.partial(
    pl.kernel,
    mesh=_mesh,
    out_type=jax.ShapeDtypeStruct((_N, _D), jnp.float32),
    scratch_types=[
        pltpu.VMEM((_NCH, _CH), jnp.int32),
        pltpu.VMEM((_ROWS, _D), jnp.float32),
        pltpu.VMEM((_ROWS, _D), jnp.float32),
        pltpu.SemaphoreType.DMA,
        pltpu.SemaphoreType.DMA,
        pltpu.SemaphoreType.DMA,
        pltpu.SemaphoreType.DMA,
    ],
)
def _gather(x_hbm, table_hbm, out_hbm, idx_all, rows0, rows1, sg0, sg1, so0, so1):
    rows = (rows0, rows1)
    sg = (sg0, sg1)
    so = (so0, so1)
    wid = lax.axis_index("s") * _NC + lax.axis_index("c")
    base = wid * _PER_W
    pltpu.sync_copy(x_hbm.at[pl.ds(wid * _NCH, _NCH)], idx_all)

    def body(p, carry):
        for b in range(_NB):
            g = p * _NB + b
            off = base + g * _ROWS

            @pl.when(g >= _NB)
            def _():
                pltpu.make_async_copy(
                    rows[b], out_hbm.at[pl.ds(off, _ROWS)], so[b]).wait()

            cps = [
                pltpu.async_copy(
                    table_hbm.at[idx_all.at[g * _G + j]],
                    rows[b].at[pl.ds(j * _CH, _CH)],
                    sg[b])
                for j in range(_G)
            ]
            for c in cps:
                c.wait()
            pltpu.async_copy(rows[b], out_hbm.at[pl.ds(off, _ROWS)], so[b])
        return carry

    lax.fori_loop(0, _NG // _NB, body, 0)
    for b in range(_NB):
        pltpu.make_async_copy(
            rows[b], out_hbm.at[pl.ds(base, _ROWS)], so[b]).wait()


def kernel(x, table):
    idx = x.reshape(_N // _CH, _CH).astype(jnp.int32)
    out = _gather(idx, table)
    return out.reshape(_B, _H, _D)
